# baseline (speedup 1.0000x reference)
import math

import jax
import jax.numpy as jnp
from jax.experimental import pallas as pl
from jax.experimental.pallas import tpu as pltpu

_NBINS = 16
_J = 17
_B = 32
_T = 4096
_N = _T - 2
_G = 8
_TC = _T // _G
_WIN_MEAN = 16
_WIN_MAX = 8
_RAD2DEG = 180.0 / math.pi


def _shl(x, k):
    if k == 0:
        return x
    pad = jnp.zeros(x.shape[:-1] + (k,), x.dtype)
    return jnp.concatenate([x[..., k:], pad], axis=-1)


def _edge_shl(x):
    return jnp.concatenate([x[:, 1:], x[:, -1:]], axis=1)


def _rhythm_kernel(x_ref, bcol_ref, peak_ref, env_ref, acc_ref):
    i = pl.program_id(0)

    mags = []
    bins = []
    for j in range(_J):
        px = x_ref[j, 0]
        py = x_ref[j, 1]
        nx = bcol_ref[0, j, 0]
        ny = bcol_ref[0, j, 1]
        mx = jnp.concatenate([px[:, 1:], nx], axis=1) - px
        my = jnp.concatenate([py[:, 1:], ny], axis=1) - py
        mags.append(jnp.sqrt(mx * mx + my * my))
        phase = jnp.arctan2(my, mx)
        r = (phase * _RAD2DEG) % 180.0
        bins.append(jnp.floor(r).astype(jnp.int32) & (_NBINS - 1))

    for b in range(_NBINS):
        s = None
        for j in range(_J):
            c = jnp.where(bins[j] == b, mags[j], 0.0)
            s = c if s is None else s + c
        acc_ref[b, :, pl.ds(i * _TC, _TC)] = s

    @pl.when(i == _G - 1)
    def _():
        rhy = jnp.zeros((_B, _T), jnp.float32)
        for b in range(_NBINS):
            d = acc_ref[b]
            sf = _edge_shl(d) - d
            rhy = rhy + jnp.maximum(sf, 0.0)
        env = rhy / jnp.max(rhy, axis=1, keepdims=True)
        gm = jnp.sum(env, axis=1, keepdims=True) / float(_N)

        t_idx = jax.lax.broadcasted_iota(jnp.int32, (_B, _T), 1)
        s2 = env + _shl(env, 1)
        s4 = s2 + _shl(s2, 2)
        s8 = s4 + _shl(s4, 4)
        s16 = s8 + _shl(s8, 8)
        lm = jnp.where(t_idx <= _N - _WIN_MEAN, s16 / float(_WIN_MEAN), 0.0)

        m = jnp.maximum(env, _shl(env, 1))
        m = jnp.maximum(m, _shl(m, 2))
        m = jnp.maximum(m, _shl(m, 4))
        lx = jnp.where(t_idx <= _N - _WIN_MAX, m, 0.0)

        peak = ((lx - lm > 0.1 * gm) & (lx == env)).astype(jnp.int32)
        dmask = (_shl(peak, 1) - peak) != 0
        dmask = dmask & (t_idx < _N - 1)
        peak_ref[...] = peak * dmask.astype(jnp.int32)
        env_ref[...] = env


@jax.jit
def _run(gxy, bcol):
    peak, env = pl.pallas_call(
        _rhythm_kernel,
        grid=(_G,),
        in_specs=[
            pl.BlockSpec((_J, 2, _B, _TC), lambda i: (0, 0, 0, i)),
            pl.BlockSpec((1, _J, 2, _B, 1), lambda i: (i, 0, 0, 0, 0)),
        ],
        out_specs=[
            pl.BlockSpec((_B, _T), lambda i: (0, 0)),
            pl.BlockSpec((_B, _T), lambda i: (0, 0)),
        ],
        out_shape=[
            jax.ShapeDtypeStruct((_B, _T), jnp.int32),
            jax.ShapeDtypeStruct((_B, _T), jnp.float32),
        ],
        scratch_shapes=[pltpu.VMEM((_NBINS, _B, _T), jnp.float32)],
    )(gxy, bcol)
    return peak[:, :_N], env[:, :_N, None]


def kernel(pose):
    gxy = jnp.transpose(pose[:, :, :, :2], (2, 3, 0, 1))
    bcol = jnp.concatenate([gxy[:, :, :, _TC::_TC], gxy[:, :, :, -1:]], axis=3)
    bcol = jnp.moveaxis(bcol, 3, 0)[..., None]
    return _run(gxy, bcol)
